# Initial kernel scaffold; baseline (speedup 1.0000x reference)
import jax
import jax.numpy as jnp
from jax.experimental import pallas as pl


def kernel(word, feat, W_word, W_feat):
    raise NotImplementedError("write your pallas kernel here")


# speedup vs baseline: 4.1892x; 4.1892x over previous
import functools

import jax
import jax.numpy as jnp
from jax import lax
from jax.experimental import pallas as pl
from jax.experimental.pallas import tpu as pltpu
from jax.experimental.pallas import tpu_sc as plsc

_DIM = 32
_CHUNK = 1024


def _build(n_rows: int):
    info = plsc.get_sparse_core_info()
    nw = info.num_cores * info.num_subcores
    assert n_rows % nw == 0
    per_w = n_rows // nw
    assert per_w % _CHUNK == 0
    n_chunks = per_w // _CHUNK

    mesh = plsc.VectorSubcoreMesh(core_axis_name="c", subcore_axis_name="s")

    @functools.partial(
        pl.kernel,
        mesh=mesh,
        out_type=jax.ShapeDtypeStruct((n_rows, 2, _DIM), jnp.float32),
        compiler_params=pltpu.CompilerParams(use_tc_tiling_on_sc=False),
        scratch_types=[
            pltpu.VMEM((_CHUNK,), jnp.int32),
            pltpu.VMEM((_CHUNK,), jnp.int32),
            pltpu.VMEM((_CHUNK, _DIM), jnp.float32),
            pltpu.VMEM((_CHUNK, _DIM), jnp.float32),
            pltpu.SemaphoreType.DMA,
            pltpu.SemaphoreType.DMA,
        ],
    )
    def k(ww_hbm, wf_hbm, word_hbm, feat_hbm, out_hbm,
          idx_w, idx_f, rows_w, rows_f, sem_w, sem_f):
        wid = lax.axis_index("s") * info.num_cores + lax.axis_index("c")
        base = wid * per_w

        def body(g, _):
            off = base + g * _CHUNK
            pltpu.sync_copy(word_hbm.at[pl.ds(off, _CHUNK)], idx_w)
            pltpu.sync_copy(feat_hbm.at[pl.ds(off, _CHUNK)], idx_f)
            cw = pltpu.async_copy(ww_hbm.at[idx_w], rows_w, sem_w)
            cf = pltpu.async_copy(wf_hbm.at[idx_f], rows_f, sem_f)
            cw.wait()
            cf.wait()
            pltpu.sync_copy(rows_w, out_hbm.at[pl.ds(off, _CHUNK), 0])
            pltpu.sync_copy(rows_f, out_hbm.at[pl.ds(off, _CHUNK), 1])
            return _

        lax.fori_loop(0, n_chunks, body, None)

    return k


def kernel(word, feat, W_word, W_feat):
    b, l = word.shape
    n_rows = b * l
    out = _build(n_rows)(W_word, W_feat, word.reshape(-1), feat.reshape(-1))
    return out.reshape(b, l, 2 * _DIM)


# speedup vs baseline: 4.3358x; 1.0350x over previous
import functools

import jax
import jax.numpy as jnp
from jax import lax
from jax.experimental import pallas as pl
from jax.experimental.pallas import tpu as pltpu
from jax.experimental.pallas import tpu_sc as plsc

_DIM = 32
_CHUNK = 800


def _build(n_rows: int):
    info = plsc.get_sparse_core_info()
    nw = info.num_cores * info.num_subcores
    assert n_rows % nw == 0
    per_w = n_rows // nw
    assert per_w % _CHUNK == 0
    n_chunks = per_w // _CHUNK
    assert n_chunks % 2 == 0 and n_chunks >= 4

    mesh = plsc.VectorSubcoreMesh(core_axis_name="c", subcore_axis_name="s")

    @functools.partial(
        pl.kernel,
        mesh=mesh,
        out_type=jax.ShapeDtypeStruct((n_rows, 2, _DIM), jnp.float32),
        compiler_params=pltpu.CompilerParams(use_tc_tiling_on_sc=False),
        scratch_types=[
            pltpu.VMEM((2, _CHUNK), jnp.int32),
            pltpu.VMEM((2, _CHUNK), jnp.int32),
            pltpu.VMEM((2, _CHUNK, _DIM), jnp.float32),
            pltpu.VMEM((2, _CHUNK, _DIM), jnp.float32),
        ] + [pltpu.SemaphoreType.DMA] * 6,
    )
    def k(ww_hbm, wf_hbm, word_hbm, feat_hbm, out_hbm,
          idx_w, idx_f, rows_w, rows_f, si0, si1, sg0, sg1, so0, so1):
        wid = lax.axis_index("s") * info.num_cores + lax.axis_index("c")
        base = wid * per_w
        si = (si0, si1)
        sg = (sg0, sg1)
        so = (so0, so1)

        def idx_copies(g, b):
            off = base + g * _CHUNK
            return (
                pltpu.make_async_copy(
                    word_hbm.at[pl.ds(off, _CHUNK)], idx_w.at[b], si[b]),
                pltpu.make_async_copy(
                    feat_hbm.at[pl.ds(off, _CHUNK)], idx_f.at[b], si[b]),
            )

        def gather_copies(b):
            return (
                pltpu.make_async_copy(ww_hbm.at[idx_w.at[b]], rows_w.at[b], sg[b]),
                pltpu.make_async_copy(wf_hbm.at[idx_f.at[b]], rows_f.at[b], sg[b]),
            )

        def write_copies(g, b):
            off = base + g * _CHUNK
            return (
                pltpu.make_async_copy(
                    rows_w.at[b], out_hbm.at[pl.ds(off, _CHUNK), 0], so[b]),
                pltpu.make_async_copy(
                    rows_f.at[b], out_hbm.at[pl.ds(off, _CHUNK), 1], so[b]),
            )

        for c in idx_copies(0, 0):
            c.start()

        def chunk_step(g, b):
            @pl.when(g >= 2)
            def _():
                for c in write_copies(g - 2, b):
                    c.wait()
            for c in idx_copies(g, b):
                c.wait()
            for c in gather_copies(b):
                c.start()
            @pl.when(g + 1 < n_chunks)
            def _():
                for c in idx_copies(g + 1, 1 - b):
                    c.start()
            for c in gather_copies(b):
                c.wait()
            for c in write_copies(g, b):
                c.start()

        @pl.loop(0, n_chunks, step=2)
        def _(t):
            chunk_step(t, 0)
            chunk_step(t + 1, 1)

        for c in write_copies(n_chunks - 2, 0):
            c.wait()
        for c in write_copies(n_chunks - 1, 1):
            c.wait()

    return k


def kernel(word, feat, W_word, W_feat):
    b, l = word.shape
    n_rows = b * l
    out = _build(n_rows)(W_word, W_feat, word.reshape(-1), feat.reshape(-1))
    return out.reshape(b, l, 2 * _DIM)
